# baseline (speedup 1.0000x reference)
import functools

import jax
import jax.numpy as jnp
from jax import lax
from jax.experimental import pallas as pl
from jax.experimental.pallas import tpu as pltpu
from jax.experimental.pallas import tpu_sc as plsc

N_FIELDS = 26
BATCH = 16384
VOCAB = 100000
FEAT = 16
L = 16
GATHER_ROWS = 128
VTILES = 98
VPAD = VTILES * 8 * 128


def _rowsum_call(nf, off):
    def body(x_ref, o_ref):
        s = jnp.sum(x_ref[...], axis=1, keepdims=True)
        col = lax.broadcasted_iota(jnp.int32, (1, 1, VOCAB), 2)
        s = jnp.where(col == 0, 0.0, s)
        s = jnp.concatenate(
            [s, jnp.zeros((1, 1, VPAD - VOCAB), jnp.float32)], axis=2)
        o_ref[...] = s.reshape(1, VTILES, 8, 128)

    return pl.pallas_call(
        body,
        grid=(nf,),
        in_specs=[pl.BlockSpec((1, FEAT, VOCAB), lambda f: (f + off, 0, 0))],
        out_specs=pl.BlockSpec((1, VTILES, 8, 128), lambda f: (f, 0, 0, 0)),
        out_shape=jax.ShapeDtypeStruct((nf, VTILES, 8, 128), jnp.float32),
    )


def _gather_call(nf, with_base):
    info = plsc.get_sparse_core_info()
    nc, ns = info.num_cores, info.num_subcores
    nw = nc * ns
    bpw = BATCH // nw
    nchunk = bpw // GATHER_ROWS
    nrows = nf * nchunk
    cpf = bpw // L
    mesh = plsc.VectorSubcoreMesh(core_axis_name="c", subcore_axis_name="s")

    @functools.partial(
        pl.kernel,
        out_type=jax.ShapeDtypeStruct((BATCH,), jnp.float32),
        mesh=mesh,
        compiler_params=pltpu.CompilerParams(
            needs_layout_passes=False, use_tc_tiling_on_sc=False),
        scratch_types=[
            pltpu.VMEM((nf, bpw), jnp.int32),
            pltpu.VMEM((nrows, GATHER_ROWS), jnp.int32),
            pltpu.VMEM((bpw,), jnp.float32),
            pltpu.VMEM((GATHER_ROWS,), jnp.float32),
            pltpu.VMEM((GATHER_ROWS,), jnp.float32),
            pltpu.VMEM((bpw,), jnp.float32),
            pltpu.VMEM((L,), jnp.float32),
            pltpu.VMEM_SHARED((nf * VPAD,), jnp.float32),
            pltpu.SemaphoreType.DMA,
            pltpu.SemaphoreType.DMA,
            pltpu.SemaphoreType.DMA,
        ],
    )
    def sc_call(*refs):
        x_refs = refs[:nf]
        s_ref, bias_ref = refs[nf], refs[nf + 1]
        pos = nf + 2
        base_ref = refs[pos] if with_base else None
        pos += 1 if with_base else 0
        out_ref = refs[pos]
        (raw, gidx, acc, buf0, buf1, out_v, bias_v, s_spmem,
         sem_i, sem0, sem1) = refs[pos + 1:]

        sid = lax.axis_index("s")
        wid = sid * nc + lax.axis_index("c")
        base = wid * bpw

        copies = [
            pltpu.async_copy(x_refs[f].at[pl.ds(base, bpw)], raw.at[f], sem_i)
            for f in range(nf)
        ]
        copies.append(pltpu.async_copy(bias_ref, bias_v, sem_i))
        if with_base:
            copies.append(
                pltpu.async_copy(base_ref.at[pl.ds(base, bpw)], acc, sem_i))
        spw = nf * VPAD // ns
        pltpu.sync_copy(s_ref.at[pl.ds(sid * spw, spw)],
                        s_spmem.at[pl.ds(sid * spw, spw)])
        for c in copies:
            c.wait()

        if not with_base:
            @plsc.parallel_loop(0, cpf, unroll=4)
            def _(c):
                acc[pl.ds(c * L, L)] = jnp.zeros((L,), jnp.float32)

        for f in range(nf):
            @plsc.parallel_loop(0, cpf, unroll=4)
            def _stage(c):
                vec = raw[f, pl.ds(c * L, L)]
                gidx[f * nchunk + (c >> 3), pl.ds((c & 7) * L, L)] = (
                    vec + f * VPAD)

        plsc.subcore_barrier()

        def accum(buf, r):
            aoff = (r & (nchunk - 1)) * GATHER_ROWS

            @plsc.parallel_loop(0, GATHER_ROWS // L, unroll=4)
            def _(i):
                plsc.addupdate(
                    acc.at[pl.ds(aoff + i * L, L)], buf[pl.ds(i * L, L)])

        pltpu.async_copy(s_spmem.at[gidx.at[0]], buf0, sem0)

        def step(k, _):
            r0 = 2 * k
            pltpu.async_copy(s_spmem.at[gidx.at[r0 + 1]], buf1, sem1)
            pltpu.make_async_copy(s_spmem.at[gidx.at[r0]], buf0, sem0).wait()
            accum(buf0, r0)
            r2 = jnp.minimum(r0 + 2, nrows - 1)
            pltpu.async_copy(s_spmem.at[gidx.at[r2]], buf0, sem0)
            pltpu.make_async_copy(
                s_spmem.at[gidx.at[r0 + 1]], buf1, sem1).wait()
            accum(buf1, r0 + 1)
            return 0

        lax.fori_loop(0, nrows // 2, step, 0)
        pltpu.make_async_copy(
            s_spmem.at[gidx.at[nrows - 1]], buf0, sem0).wait()

        bias_vec = bias_v[...]

        @plsc.parallel_loop(0, cpf, unroll=4)
        def _(c):
            sl = pl.ds(c * L, L)
            out_v[sl] = acc[sl] + bias_vec

        pltpu.sync_copy(out_v, out_ref.at[pl.ds(base, bpw)])

    return sc_call


def kernel(x_0, x_1, x_2, x_3, x_4, x_5, x_6, x_7, x_8, x_9, x_10, x_11,
           x_12, x_13, x_14, x_15, x_16, x_17, x_18, x_19, x_20, x_21, x_22,
           x_23, x_24, x_25, tables, bias):
    xs = (x_0, x_1, x_2, x_3, x_4, x_5, x_6, x_7, x_8, x_9, x_10, x_11,
          x_12, x_13, x_14, x_15, x_16, x_17, x_18, x_19, x_20, x_21, x_22,
          x_23, x_24, x_25)
    chunks = (2, 7, 8, 6, 3)
    tt = jnp.transpose(tables, (0, 2, 1))
    bias_splat = jnp.broadcast_to(jnp.reshape(bias, ()), (L,))
    zeros_splat = jnp.zeros((L,), jnp.float32)
    out = None
    off = 0
    for nf in chunks:
        s = _rowsum_call(nf, off)(tt).reshape(nf * VPAD)
        if out is None:
            out = _gather_call(nf, False)(*xs[:nf], s, bias_splat)
        else:
            out = _gather_call(nf, True)(
                *xs[off:off + nf], s, zeros_splat, out)
        off += nf
    return out.reshape(BATCH, 1)


# speedup vs baseline: 1.0311x; 1.0311x over previous
import functools

import jax
import jax.numpy as jnp
from jax import lax
from jax.experimental import pallas as pl
from jax.experimental.pallas import tpu as pltpu
from jax.experimental.pallas import tpu_sc as plsc

N_FIELDS = 26
BATCH = 16384
VOCAB = 100000
FEAT = 16
L = 16
GATHER_ROWS = 128
VTILES = 98
VPAD = VTILES * 8 * 128


def _rowsum_call(nf, off):
    def body(x_ref, o_ref):
        s = jnp.sum(x_ref[...], axis=1, keepdims=True)
        col = lax.broadcasted_iota(jnp.int32, (1, 1, VOCAB), 2)
        s = jnp.where(col == 0, 0.0, s)
        s = jnp.concatenate(
            [s, jnp.zeros((1, 1, VPAD - VOCAB), jnp.float32)], axis=2)
        o_ref[...] = s.reshape(1, VTILES, 8, 128)

    return pl.pallas_call(
        body,
        grid=(nf,),
        in_specs=[pl.BlockSpec((1, FEAT, VOCAB), lambda f: (f + off, 0, 0))],
        out_specs=pl.BlockSpec((1, VTILES, 8, 128), lambda f: (f, 0, 0, 0)),
        out_shape=jax.ShapeDtypeStruct((nf, VTILES, 8, 128), jnp.float32),
    )


def _gather_call(nf, with_base):
    info = plsc.get_sparse_core_info()
    nc, ns = info.num_cores, info.num_subcores
    nw = nc * ns
    bpw = BATCH // nw
    nchunk = bpw // GATHER_ROWS
    nrows = nf * nchunk
    cpf = bpw // L
    mesh = plsc.VectorSubcoreMesh(core_axis_name="c", subcore_axis_name="s")

    @functools.partial(
        pl.kernel,
        out_type=jax.ShapeDtypeStruct((BATCH,), jnp.float32),
        mesh=mesh,
        compiler_params=pltpu.CompilerParams(
            needs_layout_passes=False, use_tc_tiling_on_sc=False),
        scratch_types=[
            pltpu.VMEM((nf, bpw), jnp.int32),
            pltpu.VMEM((nrows, GATHER_ROWS), jnp.int32),
            pltpu.VMEM((bpw,), jnp.float32),
            pltpu.VMEM((GATHER_ROWS,), jnp.float32),
            pltpu.VMEM((GATHER_ROWS,), jnp.float32),
            pltpu.VMEM((bpw,), jnp.float32),
            pltpu.VMEM((L,), jnp.float32),
            pltpu.VMEM_SHARED((nf * VPAD,), jnp.float32),
            pltpu.SemaphoreType.DMA,
            pltpu.SemaphoreType.DMA,
            pltpu.SemaphoreType.DMA,
        ],
    )
    def sc_call(*refs):
        x_refs = refs[:nf]
        s_ref, bias_ref = refs[nf], refs[nf + 1]
        pos = nf + 2
        base_ref = refs[pos] if with_base else None
        pos += 1 if with_base else 0
        out_ref = refs[pos]
        (raw, gidx, acc, buf0, buf1, out_v, bias_v, s_spmem,
         sem_i, sem0, sem1) = refs[pos + 1:]

        sid = lax.axis_index("s")
        wid = sid * nc + lax.axis_index("c")
        base = wid * bpw

        copies = [
            pltpu.async_copy(x_refs[f].at[pl.ds(base, bpw)], raw.at[f], sem_i)
            for f in range(nf)
        ]
        copies.append(pltpu.async_copy(bias_ref, bias_v, sem_i))
        if with_base:
            copies.append(
                pltpu.async_copy(base_ref.at[pl.ds(base, bpw)], acc, sem_i))
        spw = nf * VPAD // ns
        pltpu.sync_copy(s_ref.at[pl.ds(sid * spw, spw)],
                        s_spmem.at[pl.ds(sid * spw, spw)])
        for c in copies:
            c.wait()

        if not with_base:
            @plsc.parallel_loop(0, cpf, unroll=4)
            def _(c):
                acc[pl.ds(c * L, L)] = jnp.zeros((L,), jnp.float32)

        for f in range(nf):
            @plsc.parallel_loop(0, cpf, unroll=4)
            def _stage(c):
                vec = raw[f, pl.ds(c * L, L)]
                gidx[f * nchunk + (c >> 3), pl.ds((c & 7) * L, L)] = (
                    vec + f * VPAD)

        plsc.subcore_barrier()

        def accum(buf, r):
            aoff = (r & (nchunk - 1)) * GATHER_ROWS

            @plsc.parallel_loop(0, GATHER_ROWS // L, unroll=4)
            def _(i):
                plsc.addupdate(
                    acc.at[pl.ds(aoff + i * L, L)], buf[pl.ds(i * L, L)])

        pltpu.async_copy(s_spmem.at[gidx.at[0]], buf0, sem0)

        def step(k, _):
            r0 = 2 * k
            pltpu.async_copy(s_spmem.at[gidx.at[r0 + 1]], buf1, sem1)
            pltpu.make_async_copy(s_spmem.at[gidx.at[r0]], buf0, sem0).wait()
            accum(buf0, r0)
            r2 = jnp.minimum(r0 + 2, nrows - 1)
            pltpu.async_copy(s_spmem.at[gidx.at[r2]], buf0, sem0)
            pltpu.make_async_copy(
                s_spmem.at[gidx.at[r0 + 1]], buf1, sem1).wait()
            accum(buf1, r0 + 1)
            return 0

        lax.fori_loop(0, nrows // 2, step, 0)
        pltpu.make_async_copy(
            s_spmem.at[gidx.at[nrows - 1]], buf0, sem0).wait()

        bias_vec = bias_v[...]

        @plsc.parallel_loop(0, cpf, unroll=4)
        def _(c):
            sl = pl.ds(c * L, L)
            out_v[sl] = acc[sl] + bias_vec

        pltpu.sync_copy(out_v, out_ref.at[pl.ds(base, bpw)])

    return sc_call


def kernel(x_0, x_1, x_2, x_3, x_4, x_5, x_6, x_7, x_8, x_9, x_10, x_11,
           x_12, x_13, x_14, x_15, x_16, x_17, x_18, x_19, x_20, x_21, x_22,
           x_23, x_24, x_25, tables, bias):
    xs = (x_0, x_1, x_2, x_3, x_4, x_5, x_6, x_7, x_8, x_9, x_10, x_11,
          x_12, x_13, x_14, x_15, x_16, x_17, x_18, x_19, x_20, x_21, x_22,
          x_23, x_24, x_25)
    chunks = (3, 9, 9, 5)
    tt = jnp.transpose(tables, (0, 2, 1))
    bias_splat = jnp.broadcast_to(jnp.reshape(bias, ()), (L,))
    zeros_splat = jnp.zeros((L,), jnp.float32)
    out = None
    off = 0
    for nf in chunks:
        s = _rowsum_call(nf, off)(tt).reshape(nf * VPAD)
        if out is None:
            out = _gather_call(nf, False)(*xs[:nf], s, bias_splat)
        else:
            out = _gather_call(nf, True)(
                *xs[off:off + nf], s, zeros_splat, out)
        off += nf
    return out.reshape(BATCH, 1)


# speedup vs baseline: 1.0336x; 1.0025x over previous
import functools

import jax
import jax.numpy as jnp
from jax import lax
from jax.experimental import pallas as pl
from jax.experimental.pallas import tpu as pltpu
from jax.experimental.pallas import tpu_sc as plsc

N_FIELDS = 26
BATCH = 16384
VOCAB = 100000
FEAT = 16
L = 16
GATHER_ROWS = 128
VTILES = 98
VPAD = VTILES * 8 * 128
HALF = VPAD // 2


def _rowsum_call(nf, off):
    def body(x_ref, o_ref):
        s = jnp.sum(x_ref[...], axis=1, keepdims=True)
        col = lax.broadcasted_iota(jnp.int32, (1, 1, VOCAB), 2)
        s = jnp.where(col == 0, 0.0, s)
        s = jnp.concatenate(
            [s, jnp.zeros((1, 1, VPAD - VOCAB), jnp.float32)], axis=2)
        u = lax.bitcast_convert_type(s, jnp.int32)
        r = u + 0x7FFF + ((u >> 16) & 1)
        lo = r[:, :, :HALF]
        hi = r[:, :, HALF:]
        word = ((lo >> 16) & 0xFFFF) | (hi & jnp.int32(-65536))
        o_ref[...] = word.reshape(1, VTILES // 2, 8, 128)

    return pl.pallas_call(
        body,
        grid=(nf,),
        in_specs=[pl.BlockSpec((1, FEAT, VOCAB), lambda f: (f + off, 0, 0))],
        out_specs=pl.BlockSpec((1, VTILES // 2, 8, 128), lambda f: (f, 0, 0, 0)),
        out_shape=jax.ShapeDtypeStruct((nf, VTILES // 2, 8, 128), jnp.int32),
    )


def _gather_call(nf, with_base):
    info = plsc.get_sparse_core_info()
    nc, ns = info.num_cores, info.num_subcores
    nw = nc * ns
    bpw = BATCH // nw
    nchunk = bpw // GATHER_ROWS
    nrows = nf * nchunk
    cpf = bpw // L
    mesh = plsc.VectorSubcoreMesh(core_axis_name="c", subcore_axis_name="s")

    @functools.partial(
        pl.kernel,
        out_type=jax.ShapeDtypeStruct((BATCH,), jnp.float32),
        mesh=mesh,
        compiler_params=pltpu.CompilerParams(
            needs_layout_passes=False, use_tc_tiling_on_sc=False),
        scratch_types=[
            pltpu.VMEM((nf, bpw), jnp.int32),
            pltpu.VMEM((nrows, GATHER_ROWS), jnp.int32),
            pltpu.VMEM((nrows * GATHER_ROWS,), jnp.int32),
            pltpu.VMEM((bpw,), jnp.float32),
            pltpu.VMEM((GATHER_ROWS,), jnp.int32),
            pltpu.VMEM((GATHER_ROWS,), jnp.int32),
            pltpu.VMEM((bpw,), jnp.float32),
            pltpu.VMEM((L,), jnp.float32),
            pltpu.VMEM_SHARED((nf * HALF,), jnp.int32),
            pltpu.SemaphoreType.DMA,
            pltpu.SemaphoreType.DMA,
            pltpu.SemaphoreType.DMA,
        ],
    )
    def sc_call(*refs):
        x_refs = refs[:nf]
        s_ref, bias_ref = refs[nf], refs[nf + 1]
        pos = nf + 2
        base_ref = refs[pos] if with_base else None
        pos += 1 if with_base else 0
        out_ref = refs[pos]
        (raw, gidx, selm, acc, buf0, buf1, out_v, bias_v, s_spmem,
         sem_i, sem0, sem1) = refs[pos + 1:]

        sid = lax.axis_index("s")
        wid = sid * nc + lax.axis_index("c")
        base = wid * bpw

        copies = [
            pltpu.async_copy(x_refs[f].at[pl.ds(base, bpw)], raw.at[f], sem_i)
            for f in range(nf)
        ]
        copies.append(pltpu.async_copy(bias_ref, bias_v, sem_i))
        if with_base:
            copies.append(
                pltpu.async_copy(base_ref.at[pl.ds(base, bpw)], acc, sem_i))
        spw = nf * HALF // ns
        pltpu.sync_copy(s_ref.at[pl.ds(sid * spw, spw)],
                        s_spmem.at[pl.ds(sid * spw, spw)])
        for c in copies:
            c.wait()

        if not with_base:
            @plsc.parallel_loop(0, cpf, unroll=4)
            def _(c):
                acc[pl.ds(c * L, L)] = jnp.zeros((L,), jnp.float32)

        for f in range(nf):
            @plsc.parallel_loop(0, cpf, unroll=4)
            def _stage(c):
                vec = raw[f, pl.ds(c * L, L)]
                ishi = vec >= HALF
                pair = jnp.where(ishi, vec - HALF, vec) + f * HALF
                row = f * nchunk + (c >> 3)
                gidx[row, pl.ds((c & 7) * L, L)] = pair
                selm[pl.ds(row * GATHER_ROWS + (c & 7) * L, L)] = (
                    jnp.where(ishi, 1, 0))

        plsc.subcore_barrier()

        def accum(buf, r):
            aoff = (r & (nchunk - 1)) * GATHER_ROWS

            @plsc.parallel_loop(0, GATHER_ROWS // L, unroll=4)
            def _(i):
                word = buf[pl.ds(i * L, L)]
                sel = selm[pl.ds(r * GATHER_ROWS + i * L, L)]
                bits = jnp.where(
                    sel == 0, word << 16, word & jnp.int32(-65536))
                plsc.addupdate(
                    acc.at[pl.ds(aoff + i * L, L)],
                    plsc.bitcast(bits, jnp.float32))

        pltpu.async_copy(s_spmem.at[gidx.at[0]], buf0, sem0)

        def step(k, _):
            r0 = 2 * k
            pltpu.async_copy(s_spmem.at[gidx.at[r0 + 1]], buf1, sem1)
            pltpu.make_async_copy(s_spmem.at[gidx.at[r0]], buf0, sem0).wait()
            accum(buf0, r0)
            r2 = jnp.minimum(r0 + 2, nrows - 1)
            pltpu.async_copy(s_spmem.at[gidx.at[r2]], buf0, sem0)
            pltpu.make_async_copy(
                s_spmem.at[gidx.at[r0 + 1]], buf1, sem1).wait()
            accum(buf1, r0 + 1)
            return 0

        lax.fori_loop(0, nrows // 2, step, 0)
        pltpu.make_async_copy(
            s_spmem.at[gidx.at[nrows - 1]], buf0, sem0).wait()

        bias_vec = bias_v[...]

        @plsc.parallel_loop(0, cpf, unroll=4)
        def _(c):
            sl = pl.ds(c * L, L)
            out_v[sl] = acc[sl] + bias_vec

        pltpu.sync_copy(out_v, out_ref.at[pl.ds(base, bpw)])

    return sc_call


def kernel(x_0, x_1, x_2, x_3, x_4, x_5, x_6, x_7, x_8, x_9, x_10, x_11,
           x_12, x_13, x_14, x_15, x_16, x_17, x_18, x_19, x_20, x_21, x_22,
           x_23, x_24, x_25, tables, bias):
    xs = (x_0, x_1, x_2, x_3, x_4, x_5, x_6, x_7, x_8, x_9, x_10, x_11,
          x_12, x_13, x_14, x_15, x_16, x_17, x_18, x_19, x_20, x_21, x_22,
          x_23, x_24, x_25)
    chunks = (3, 9, 9, 5)
    tt = jnp.transpose(tables, (0, 2, 1))
    bias_splat = jnp.broadcast_to(jnp.reshape(bias, ()), (L,))
    zeros_splat = jnp.zeros((L,), jnp.float32)
    out = None
    off = 0
    for nf in chunks:
        s = _rowsum_call(nf, off)(tt).reshape(nf * HALF)
        if out is None:
            out = _gather_call(nf, False)(*xs[:nf], s, bias_splat)
        else:
            out = _gather_call(nf, True)(
                *xs[off:off + nf], s, zeros_splat, out)
        off += nf
    return out.reshape(BATCH, 1)
